# baseline (speedup 1.0000x reference)
import functools

import jax
import jax.numpy as jnp
from jax import lax
from jax.experimental import pallas as pl
from jax.experimental.pallas import tpu as pltpu
from jax.experimental.pallas import tpu_sc as plsc

N = 10000
E = 160000
DEG = 16
NH = 128
NHW = NH // 2
IN = 4

NC = 2
NS = 16
NW = NC * NS

_sc_mesh = plsc.VectorSubcoreMesh(core_axis_name="c", subcore_axis_name="s")

EPW = E // NW
S1_C = 400
S1_CHUNKS = [(i * S1_C, S1_C) for i in range(EPW // S1_C)]
if EPW % S1_C:
    S1_CHUNKS.append((EPW - EPW % S1_C, EPW % S1_C))


@functools.partial(
    pl.kernel,
    out_type=jax.ShapeDtypeStruct((E, NHW), jnp.int32),
    mesh=_sc_mesh,
    compiler_params=pltpu.CompilerParams(use_tc_tiling_on_sc=False, needs_layout_passes=False),
    scratch_types=[
        pltpu.VMEM((S1_C,), jnp.int32),
        pltpu.VMEM((S1_C,), jnp.int32),
        pltpu.VMEM((S1_C,), jnp.int32),
        pltpu.VMEM((S1_C,), jnp.int32),
        pltpu.VMEM((S1_C, NHW), jnp.int32),
        pltpu.VMEM((S1_C, NHW), jnp.int32),
        pltpu.VMEM((S1_C, NHW), jnp.int32),
        pltpu.VMEM((S1_C, NHW), jnp.int32),
        [pltpu.SemaphoreType.DMA] * 6,
    ],
)
def _s1_gather(a_hbm, b_hbm, recv_hbm, send_hbm, out_hbm,
               idx_r0, idx_r1, idx_s0, idx_s1, ra0, ra1, rb0, rb1, sems):
    si0, si1, sg0, sg1, so0, so1 = sems
    wid = lax.axis_index("s") * NC + lax.axis_index("c")
    base = wid * EPW
    idx_r = (idx_r0, idx_r1)
    idx_s = (idx_s0, idx_s1)
    ra = (ra0, ra1)
    rb = (rb0, rb1)
    si = (si0, si1)
    sg = (sg0, sg1)
    so = (so0, so1)
    nck = len(S1_CHUNKS)
    d_i = [None] * nck
    d_g = [None] * nck
    d_o = [None] * nck

    def issue_idx(j, s):
        off, c = S1_CHUNKS[j]
        return (
            pltpu.async_copy(recv_hbm.at[pl.ds(base + off, c)],
                             idx_r[s].at[pl.ds(0, c)], si[s]),
            pltpu.async_copy(send_hbm.at[pl.ds(base + off, c)],
                             idx_s[s].at[pl.ds(0, c)], si[s]),
        )

    def issue_gathers(j, s):
        c = S1_CHUNKS[j][1]
        return (
            pltpu.async_copy(a_hbm.at[idx_r[s].at[pl.ds(0, c)]],
                             ra[s].at[pl.ds(0, c)], sg[s]),
            pltpu.async_copy(b_hbm.at[idx_s[s].at[pl.ds(0, c)]],
                             rb[s].at[pl.ds(0, c)], sg[s]),
        )

    for j in range(nck):
        s = j % 2
        s2 = 1 - s
        off, cj = S1_CHUNKS[j]
        if j == 0:
            d_i[0] = issue_idx(0, 0)
            d_i[0][0].wait()
            d_i[0][1].wait()
            d_g[0] = issue_gathers(0, 0)
            if nck > 1:
                d_i[1] = issue_idx(1, 1)
        d_g[j][0].wait()
        d_g[j][1].wait()
        if j + 1 < nck:
            if j >= 1:
                d_o[j - 1].wait()
            d_i[j + 1][0].wait()
            d_i[j + 1][1].wait()
            d_g[j + 1] = issue_gathers(j + 1, s2)
            if j + 2 < nck:
                d_i[j + 2] = issue_idx(j + 2, s)
        rav, rbv = ra[s], rb[s]

        def add_row(r, carry):
            for c in range(NHW // 16):
                sl = pl.ds(c * 16, 16)
                va = plsc.bitcast(rav[r, sl], jnp.bfloat16)
                vb = plsc.bitcast(rbv[r, sl], jnp.bfloat16)
                rav[r, sl] = plsc.bitcast(va + vb, jnp.int32)
            return carry

        lax.fori_loop(0, cj, add_row, 0)
        d_o[j] = pltpu.async_copy(
            ra[s].at[pl.ds(0, cj)],
            out_hbm.at[pl.ds(base + off, cj)], so[s])
    if nck >= 2:
        d_o[nck - 2].wait()
    d_o[nck - 1].wait()


NPAD = 10240
NPW = NPAD // NW
S2_NN = 40
S2_NCHUNK = NPW // S2_NN
S2_R = S2_NN * DEG


@functools.partial(
    pl.kernel,
    out_type=jax.ShapeDtypeStruct((NPAD, NHW), jnp.int32),
    mesh=_sc_mesh,
    compiler_params=pltpu.CompilerParams(use_tc_tiling_on_sc=False, needs_layout_passes=False),
    scratch_types=[
        pltpu.VMEM((S2_R,), jnp.int32),
        pltpu.VMEM((S2_R,), jnp.int32),
        pltpu.VMEM((S2_R, NHW), jnp.int32),
        pltpu.VMEM((S2_R, NHW), jnp.int32),
        pltpu.VMEM((S2_NN, NHW), jnp.int32),
        pltpu.VMEM((S2_NN, NHW), jnp.int32),
        [pltpu.SemaphoreType.DMA] * 6,
    ],
)
def _s2_aggregate(msgs_hbm, e2n_hbm, out_hbm,
                  idx0, idx1, rows0, rows1, acc0, acc1, sems):
    si0, si1, sg0, sg1, so0, so1 = sems
    wid = lax.axis_index("s") * NC + lax.axis_index("c")
    nbase = wid * NPW
    idx = (idx0, idx1)
    rows = (rows0, rows1)
    acc = (acc0, acc1)
    si = (si0, si1)
    sg = (sg0, sg1)
    so = (so0, so1)
    d_i = [None] * S2_NCHUNK
    d_g = [None] * S2_NCHUNK
    d_o = [None] * S2_NCHUNK

    def issue_idx(j, s):
        return pltpu.async_copy(
            e2n_hbm.at[pl.ds((nbase + j * S2_NN) * DEG, S2_R)], idx[s], si[s])

    for j in range(S2_NCHUNK):
        s = j % 2
        s2 = 1 - s
        if j == 0:
            d_i[0] = issue_idx(0, 0)
            d_i[0].wait()
            d_g[0] = pltpu.async_copy(msgs_hbm.at[idx0], rows0, sg[0])
            if S2_NCHUNK > 1:
                d_i[1] = issue_idx(1, 1)
        d_g[j].wait()
        if j + 1 < S2_NCHUNK:
            if j >= 1:
                d_o[j - 1].wait()
            d_i[j + 1].wait()
            d_g[j + 1] = pltpu.async_copy(msgs_hbm.at[idx[s2]], rows[s2], sg[s2])
            if j + 2 < S2_NCHUNK:
                d_i[j + 2] = issue_idx(j + 2, s)
        rv = rows[s]
        av = acc[s]

        def node_sum(i, carry):
            def k_body(k, accs):
                return tuple(
                    accs[c] + plsc.bitcast(rv[i * DEG + k, pl.ds(c * 16, 16)],
                                           jnp.bfloat16)
                    for c in range(NHW // 16)
                )

            zeros = tuple(jnp.zeros((32,), jnp.bfloat16) for _ in range(NHW // 16))
            accs = lax.fori_loop(0, DEG, k_body, zeros)
            for c in range(NHW // 16):
                av[i, pl.ds(c * 16, 16)] = plsc.bitcast(accs[c], jnp.int32)
            return carry

        lax.fori_loop(0, S2_NN, node_sum, 0)
        d_o[j] = pltpu.async_copy(
            av, out_hbm.at[pl.ds(nbase + j * S2_NN, S2_NN)], so[s])
    if S2_NCHUNK >= 2:
        d_o[S2_NCHUNK - 2].wait()
    d_o[S2_NCHUNK - 1].wait()


P1_BN = 400


def _p1_body(h_ref, w1rT_ref, w1sT_ref, b1_ref, a_ref, b_ref):
    h = h_ref[...]
    a = jnp.dot(h, w1rT_ref[...], preferred_element_type=jnp.float32)
    b = jnp.dot(h, w1sT_ref[...], preferred_element_type=jnp.float32) + b1_ref[...]
    a_ref[...] = a.astype(jnp.bfloat16)
    b_ref[...] = b.astype(jnp.bfloat16)


def _p1(h, w1rT, w1sT, b1):
    grid = N // P1_BN
    return pl.pallas_call(
        _p1_body,
        grid=(grid,),
        in_specs=[
            pl.BlockSpec((P1_BN, NH), lambda i: (i, 0)),
            pl.BlockSpec((NH, NH), lambda i: (0, 0)),
            pl.BlockSpec((NH, NH), lambda i: (0, 0)),
            pl.BlockSpec((1, NH), lambda i: (0, 0)),
        ],
        out_specs=[
            pl.BlockSpec((P1_BN, NH), lambda i: (i, 0)),
            pl.BlockSpec((P1_BN, NH), lambda i: (i, 0)),
        ],
        out_shape=[
            jax.ShapeDtypeStruct((N, NH), jnp.bfloat16),
            jax.ShapeDtypeStruct((N, NH), jnp.bfloat16),
        ],
    )(h, w1rT, w1sT, b1)


P2_BE = 2000


def _p2_body(pre1_ref, e1_ref, w2T_ref, b2_ref, out_ref):
    msg = jnp.tanh(pre1_ref[...].astype(jnp.float32))
    msg = jnp.dot(msg, w2T_ref[...], preferred_element_type=jnp.float32) + b2_ref[...]
    out_ref[...] = (jnp.tanh(msg) * e1_ref[...]).astype(jnp.bfloat16)


def _p2(pre1, e1, w2T, b2):
    grid = E // P2_BE
    return pl.pallas_call(
        _p2_body,
        grid=(grid,),
        in_specs=[
            pl.BlockSpec((P2_BE, NH), lambda i: (i, 0)),
            pl.BlockSpec((P2_BE, 1), lambda i: (i, 0)),
            pl.BlockSpec((NH, NH), lambda i: (0, 0)),
            pl.BlockSpec((1, NH), lambda i: (0, 0)),
        ],
        out_specs=pl.BlockSpec((P2_BE, NH), lambda i: (i, 0)),
        out_shape=jax.ShapeDtypeStruct((E, NH), jnp.bfloat16),
    )(pre1, e1, w2T, b2)


P3_BN = 400


def _p3_body(inc_ref, h_ref, x_ref,
             irT_ref, iiT_ref, inT_ref, ib_ref,
             hrT_ref, hiT_ref, hhT_ref,
             o1T_ref, o2T_ref, o3T_ref, ob_ref, ob3_ref,
             newh_ref, pred_ref):
    agg = inc_ref[...].astype(jnp.float32) * (1.0 / float(N - 1))
    x = x_ref[...]
    h = h_ref[...]
    inp_r = jnp.dot(x, irT_ref[...], preferred_element_type=jnp.float32) + ib_ref[0, :1, :]
    inp_i = jnp.dot(x, iiT_ref[...], preferred_element_type=jnp.float32) + ib_ref[0, 1:2, :]
    inp_n = jnp.dot(x, inT_ref[...], preferred_element_type=jnp.float32) + ib_ref[0, 2:3, :]
    r = jax.nn.sigmoid(inp_r + jnp.dot(agg, hrT_ref[...], preferred_element_type=jnp.float32))
    ii = jax.nn.sigmoid(inp_i + jnp.dot(agg, hiT_ref[...], preferred_element_type=jnp.float32))
    nn = jnp.tanh(inp_n + r * jnp.dot(agg, hhT_ref[...], preferred_element_type=jnp.float32))
    new_h = (1.0 - ii) * nn + ii * h
    newh_ref[...] = new_h
    p = jax.nn.relu(jnp.dot(new_h, o1T_ref[...], preferred_element_type=jnp.float32) + ob_ref[0, :1, :])
    p = jax.nn.relu(jnp.dot(p, o2T_ref[...], preferred_element_type=jnp.float32) + ob_ref[0, 1:2, :])
    p3 = jnp.dot(p, o3T_ref[...], preferred_element_type=jnp.float32) + ob3_ref[...]
    pred_ref[...] = x + p3


def _p3(incoming, h, x, irT, iiT, inT, ib, hrT, hiT, hhT, o1T, o2T, o3T, ob, ob3):
    grid = N // P3_BN
    full = lambda i: (0, 0)
    return pl.pallas_call(
        _p3_body,
        grid=(grid,),
        in_specs=[
            pl.BlockSpec((P3_BN, NH), lambda i: (i, 0)),
            pl.BlockSpec((P3_BN, NH), lambda i: (i, 0)),
            pl.BlockSpec((P3_BN, IN), lambda i: (i, 0)),
            pl.BlockSpec((IN, NH), full),
            pl.BlockSpec((IN, NH), full),
            pl.BlockSpec((IN, NH), full),
            pl.BlockSpec((1, 3, NH), lambda i: (0, 0, 0)),
            pl.BlockSpec((NH, NH), full),
            pl.BlockSpec((NH, NH), full),
            pl.BlockSpec((NH, NH), full),
            pl.BlockSpec((NH, NH), full),
            pl.BlockSpec((NH, NH), full),
            pl.BlockSpec((NH, IN), full),
            pl.BlockSpec((1, 2, NH), lambda i: (0, 0, 0)),
            pl.BlockSpec((1, IN), full),
        ],
        out_specs=[
            pl.BlockSpec((P3_BN, NH), lambda i: (i, 0)),
            pl.BlockSpec((P3_BN, IN), lambda i: (i, 0)),
        ],
        out_shape=[
            jax.ShapeDtypeStruct((N, NH), jnp.float32),
            jax.ShapeDtypeStruct((N, IN), jnp.float32),
        ],
    )(incoming, h, x, irT, iiT, inT, ib, hrT, hiT, hhT, o1T, o2T, o3T, ob, ob3)


def _pack_i32(x_bf):
    r = x_bf.shape[0]
    return jax.lax.bitcast_convert_type(x_bf.reshape(r, NHW, 2), jnp.int32)


def _unpack_bf(x_i32):
    r = x_i32.shape[0]
    return jax.lax.bitcast_convert_type(x_i32, jnp.bfloat16).reshape(r, NH)


def kernel(inputs, hidden, edges, node_masks, send_edges, recv_edges,
           edge2node_inds,
           msg_fc1_w, msg_fc1_b, msg_fc2_w, msg_fc2_b,
           hidden_r_w, hidden_i_w, hidden_h_w,
           input_r_w, input_r_b, input_i_w, input_i_b, input_n_w, input_n_b,
           out_fc1_w, out_fc1_b, out_fc2_w, out_fc2_b, out_fc3_w, out_fc3_b):
    h = hidden[0]
    x = inputs[0]
    e1 = edges[0, :, 1:2]

    w1rT = jnp.transpose(msg_fc1_w[1][:, :NH])
    w1sT = jnp.transpose(msg_fc1_w[1][:, NH:])
    b1 = msg_fc1_b[1][None, :]
    w2T = jnp.transpose(msg_fc2_w[1])
    b2 = msg_fc2_b[1][None, :]

    a_tab, b_tab = _p1(h, w1rT, w1sT, b1)

    pre1_i32 = _s1_gather(_pack_i32(a_tab), _pack_i32(b_tab),
                          recv_edges, send_edges)

    msgs = _p2(_unpack_bf(pre1_i32), e1, w2T, b2)

    e2n_flat = jnp.pad(edge2node_inds, ((0, NPAD - N), (0, 0))).reshape(-1)
    inc_i32 = _s2_aggregate(_pack_i32(msgs), e2n_flat)
    incoming = _unpack_bf(inc_i32)

    ib = jnp.stack([input_r_b, input_i_b, input_n_b])[None]
    ob = jnp.stack([out_fc1_b, out_fc2_b])[None]
    new_h, pred = _p3(
        incoming, h, x,
        jnp.transpose(input_r_w), jnp.transpose(input_i_w), jnp.transpose(input_n_w), ib,
        jnp.transpose(hidden_r_w), jnp.transpose(hidden_i_w), jnp.transpose(hidden_h_w),
        jnp.transpose(out_fc1_w), jnp.transpose(out_fc2_w), jnp.transpose(out_fc3_w), ob,
        out_fc3_b[None, :],
    )

    pred_all = pred[None]
    hidden_out = new_h[None]
    return (pred_all, hidden_out)


# speedup vs baseline: 2.5059x; 2.5059x over previous
import functools

import jax
import jax.numpy as jnp
from jax import lax
from jax.experimental import pallas as pl
from jax.experimental.pallas import tpu as pltpu
from jax.experimental.pallas import tpu_sc as plsc

N = 10000
E = 160000
DEG = 16
NH = 128
IN = 4

NC = 2
NS = 16
NW = NC * NS

EPW = E // NW
S1_C = 400
S1_CHUNKS = [(i * S1_C, S1_C) for i in range(EPW // S1_C)]
if EPW % S1_C:
    S1_CHUNKS.append((EPW - EPW % S1_C, EPW % S1_C))

_sc_mesh = plsc.VectorSubcoreMesh(core_axis_name="c", subcore_axis_name="s")


@functools.partial(
    pl.kernel,
    out_type=jax.ShapeDtypeStruct((E, NH), jnp.float32),
    mesh=_sc_mesh,
    scratch_types=[
        pltpu.VMEM((S1_C,), jnp.int32),
        pltpu.VMEM((S1_C,), jnp.int32),
        pltpu.VMEM((S1_C,), jnp.int32),
        pltpu.VMEM((S1_C,), jnp.int32),
        pltpu.VMEM((S1_C, NH), jnp.float32),
        pltpu.VMEM((S1_C, NH), jnp.float32),
        pltpu.SemaphoreType.DMA,
        pltpu.SemaphoreType.DMA,
        pltpu.SemaphoreType.DMA,
        pltpu.SemaphoreType.DMA,
        pltpu.SemaphoreType.DMA,
        pltpu.SemaphoreType.DMA,
    ],
)
def _s1_gather(a_hbm, b_hbm, recv_hbm, send_hbm, out_hbm,
               idx_r0, idx_r1, idx_s0, idx_s1, rows0, rows1,
               si0, si1, sg0, sg1, so0, so1):
    wid = lax.axis_index("s") * NC + lax.axis_index("c")
    base = wid * EPW
    idx_r = (idx_r0, idx_r1)
    idx_s = (idx_s0, idx_s1)
    rows = (rows0, rows1)
    si = (si0, si1)
    sg = (sg0, sg1)
    so = (so0, so1)
    nck = len(S1_CHUNKS)
    d_i = [None] * nck
    d_g = [None] * nck
    d_o = [None] * nck

    def issue_idx(j, s):
        off, c = S1_CHUNKS[j]
        return (
            pltpu.async_copy(recv_hbm.at[pl.ds(base + off, c)],
                             idx_r[s].at[pl.ds(0, c)], si[s]),
            pltpu.async_copy(send_hbm.at[pl.ds(base + off, c)],
                             idx_s[s].at[pl.ds(0, c)], si[s]),
        )

    for j in range(nck):
        s = j % 2
        s2 = 1 - s
        cj = S1_CHUNKS[j][1]
        if j == 0:
            d_i[0] = issue_idx(0, 0)
            d_i[0][0].wait()
            d_i[0][1].wait()
            d_g[0] = pltpu.async_copy(
                a_hbm.at[idx_r[0].at[pl.ds(0, cj)]],
                rows[0].at[pl.ds(0, cj)], sg[0])
            if nck > 1:
                d_i[1] = issue_idx(1, 1)
        d_g[j].wait()
        d_h = pltpu.async_copy(
            b_hbm.at[idx_s[s].at[pl.ds(0, cj)]],
            rows[s].at[pl.ds(0, cj)], sg[s], add=True)
        if j + 1 < nck:
            cn = S1_CHUNKS[j + 1][1]
            if j >= 1:
                d_o[j - 1].wait()
            d_i[j + 1][0].wait()
            d_i[j + 1][1].wait()
            d_g[j + 1] = pltpu.async_copy(
                a_hbm.at[idx_r[s2].at[pl.ds(0, cn)]],
                rows[s2].at[pl.ds(0, cn)], sg[s2])
        d_h.wait()
        if j + 2 < len(S1_CHUNKS):
            d_i[j + 2] = issue_idx(j + 2, s)
        off = S1_CHUNKS[j][0]
        d_o[j] = pltpu.async_copy(
            rows[s].at[pl.ds(0, cj)],
            out_hbm.at[pl.ds(base + off, cj)], so[s])
    if nck >= 2:
        d_o[nck - 2].wait()
    d_o[nck - 1].wait()


NPAD = 10240
NPW = NPAD // NW
S2_Q = 4
S2_QN = NPW // S2_Q


@functools.partial(
    pl.kernel,
    out_type=jax.ShapeDtypeStruct((NPAD, NH), jnp.float32),
    mesh=_sc_mesh,
    compiler_params=pltpu.CompilerParams(use_tc_tiling_on_sc=False),
    scratch_types=[
        pltpu.VMEM((DEG * NPW,), jnp.int32),
        pltpu.VMEM((NPW, NH), jnp.float32),
        pltpu.SemaphoreType.DMA,
        pltpu.SemaphoreType.DMA,
        pltpu.SemaphoreType.DMA,
        pltpu.SemaphoreType.DMA,
    ],
)
def _s2_aggregate(msgs_hbm, e2nw_hbm, out_hbm,
                  idx2, acc, sq0, sq1, sq2, sq3):
    wid = lax.axis_index("s") * NC + lax.axis_index("c")
    nbase = wid * NPW
    sq = (sq0, sq1, sq2, sq3)
    pltpu.sync_copy(e2nw_hbm.at[pl.ds(wid * DEG * NPW, DEG * NPW)], idx2)
    d_prev = [None] * S2_Q
    for k in range(DEG):
        for q in range(S2_Q):
            if k > 0:
                d_prev[q].wait()
            d_prev[q] = pltpu.async_copy(
                msgs_hbm.at[idx2.at[pl.ds(k * NPW + q * S2_QN, S2_QN)]],
                acc.at[pl.ds(q * S2_QN, S2_QN)], sq[q], add=(k > 0))
    for q in range(S2_Q):
        d_prev[q].wait()
    pltpu.sync_copy(acc, out_hbm.at[pl.ds(nbase, NPW)])


P1_BN = 400


def _p1_body(h_ref, w1rT_ref, w1sT_ref, b1_ref, a_ref, b_ref):
    h = h_ref[...]
    a_ref[...] = jnp.dot(h, w1rT_ref[...], preferred_element_type=jnp.float32)
    b_ref[...] = jnp.dot(h, w1sT_ref[...], preferred_element_type=jnp.float32) + b1_ref[...]


def _p1(h, w1rT, w1sT, b1):
    grid = N // P1_BN
    return pl.pallas_call(
        _p1_body,
        grid=(grid,),
        in_specs=[
            pl.BlockSpec((P1_BN, NH), lambda i: (i, 0)),
            pl.BlockSpec((NH, NH), lambda i: (0, 0)),
            pl.BlockSpec((NH, NH), lambda i: (0, 0)),
            pl.BlockSpec((1, NH), lambda i: (0, 0)),
        ],
        out_specs=[
            pl.BlockSpec((P1_BN, NH), lambda i: (i, 0)),
            pl.BlockSpec((P1_BN, NH), lambda i: (i, 0)),
        ],
        out_shape=[
            jax.ShapeDtypeStruct((N, NH), jnp.float32),
            jax.ShapeDtypeStruct((N, NH), jnp.float32),
        ],
    )(h, w1rT, w1sT, b1)


P2_BE = 2000


def _p2_body(pre1_ref, e1_ref, w2T_ref, b2_ref, out_ref):
    msg = jnp.tanh(pre1_ref[...])
    msg = jnp.dot(msg, w2T_ref[...], preferred_element_type=jnp.float32) + b2_ref[...]
    out_ref[...] = jnp.tanh(msg) * e1_ref[...]


def _p2(pre1, e1, w2T, b2):
    grid = E // P2_BE
    return pl.pallas_call(
        _p2_body,
        grid=(grid,),
        in_specs=[
            pl.BlockSpec((P2_BE, NH), lambda i: (i, 0)),
            pl.BlockSpec((P2_BE, 1), lambda i: (i, 0)),
            pl.BlockSpec((NH, NH), lambda i: (0, 0)),
            pl.BlockSpec((1, NH), lambda i: (0, 0)),
        ],
        out_specs=pl.BlockSpec((P2_BE, NH), lambda i: (i, 0)),
        out_shape=jax.ShapeDtypeStruct((E, NH), jnp.float32),
    )(pre1, e1, w2T, b2)


P3_BN = 400


def _p3_body(inc_ref, h_ref, x_ref,
             irT_ref, iiT_ref, inT_ref, ib_ref,
             hrT_ref, hiT_ref, hhT_ref,
             o1T_ref, o2T_ref, o3T_ref, ob_ref, ob3_ref,
             newh_ref, pred_ref):
    agg = inc_ref[...] * (1.0 / float(N - 1))
    x = x_ref[...]
    h = h_ref[...]
    inp_r = jnp.dot(x, irT_ref[...], preferred_element_type=jnp.float32) + ib_ref[0, :1, :]
    inp_i = jnp.dot(x, iiT_ref[...], preferred_element_type=jnp.float32) + ib_ref[0, 1:2, :]
    inp_n = jnp.dot(x, inT_ref[...], preferred_element_type=jnp.float32) + ib_ref[0, 2:3, :]
    r = jax.nn.sigmoid(inp_r + jnp.dot(agg, hrT_ref[...], preferred_element_type=jnp.float32))
    ii = jax.nn.sigmoid(inp_i + jnp.dot(agg, hiT_ref[...], preferred_element_type=jnp.float32))
    nn = jnp.tanh(inp_n + r * jnp.dot(agg, hhT_ref[...], preferred_element_type=jnp.float32))
    new_h = (1.0 - ii) * nn + ii * h
    newh_ref[...] = new_h
    p = jax.nn.relu(jnp.dot(new_h, o1T_ref[...], preferred_element_type=jnp.float32) + ob_ref[0, :1, :])
    p = jax.nn.relu(jnp.dot(p, o2T_ref[...], preferred_element_type=jnp.float32) + ob_ref[0, 1:2, :])
    p3 = jnp.dot(p, o3T_ref[...], preferred_element_type=jnp.float32) + ob3_ref[...]
    pred_ref[...] = x + p3


def _p3(incoming, h, x, irT, iiT, inT, ib, hrT, hiT, hhT, o1T, o2T, o3T, ob, ob3):
    grid = N // P3_BN
    full = lambda i: (0, 0)
    return pl.pallas_call(
        _p3_body,
        grid=(grid,),
        in_specs=[
            pl.BlockSpec((P3_BN, NH), lambda i: (i, 0)),
            pl.BlockSpec((P3_BN, NH), lambda i: (i, 0)),
            pl.BlockSpec((P3_BN, IN), lambda i: (i, 0)),
            pl.BlockSpec((IN, NH), full),
            pl.BlockSpec((IN, NH), full),
            pl.BlockSpec((IN, NH), full),
            pl.BlockSpec((1, 3, NH), lambda i: (0, 0, 0)),
            pl.BlockSpec((NH, NH), full),
            pl.BlockSpec((NH, NH), full),
            pl.BlockSpec((NH, NH), full),
            pl.BlockSpec((NH, NH), full),
            pl.BlockSpec((NH, NH), full),
            pl.BlockSpec((NH, IN), full),
            pl.BlockSpec((1, 2, NH), lambda i: (0, 0, 0)),
            pl.BlockSpec((1, IN), full),
        ],
        out_specs=[
            pl.BlockSpec((P3_BN, NH), lambda i: (i, 0)),
            pl.BlockSpec((P3_BN, IN), lambda i: (i, 0)),
        ],
        out_shape=[
            jax.ShapeDtypeStruct((N, NH), jnp.float32),
            jax.ShapeDtypeStruct((N, IN), jnp.float32),
        ],
    )(incoming, h, x, irT, iiT, inT, ib, hrT, hiT, hhT, o1T, o2T, o3T, ob, ob3)


def kernel(inputs, hidden, edges, node_masks, send_edges, recv_edges,
           edge2node_inds,
           msg_fc1_w, msg_fc1_b, msg_fc2_w, msg_fc2_b,
           hidden_r_w, hidden_i_w, hidden_h_w,
           input_r_w, input_r_b, input_i_w, input_i_b, input_n_w, input_n_b,
           out_fc1_w, out_fc1_b, out_fc2_w, out_fc2_b, out_fc3_w, out_fc3_b):
    h = hidden[0]
    x = inputs[0]
    e1 = edges[0, :, 1:2]

    w1rT = jnp.transpose(msg_fc1_w[1][:, :NH])
    w1sT = jnp.transpose(msg_fc1_w[1][:, NH:])
    b1 = msg_fc1_b[1][None, :]
    w2T = jnp.transpose(msg_fc2_w[1])
    b2 = msg_fc2_b[1][None, :]

    a_tab, b_tab = _p1(h, w1rT, w1sT, b1)

    pre1 = _s1_gather(a_tab, b_tab, recv_edges, send_edges)

    msgs = _p2(pre1, e1, w2T, b2)

    e2nw = (jnp.pad(edge2node_inds, ((0, NPAD - N), (0, 0)))
            .T.reshape(DEG, NW, NPW).transpose(1, 0, 2)
            .reshape(-1))
    incoming = _s2_aggregate(msgs, e2nw)

    ib = jnp.stack([input_r_b, input_i_b, input_n_b])[None]
    ob = jnp.stack([out_fc1_b, out_fc2_b])[None]
    new_h, pred = _p3(
        incoming, h, x,
        jnp.transpose(input_r_w), jnp.transpose(input_i_w), jnp.transpose(input_n_w), ib,
        jnp.transpose(hidden_r_w), jnp.transpose(hidden_i_w), jnp.transpose(hidden_h_w),
        jnp.transpose(out_fc1_w), jnp.transpose(out_fc2_w), jnp.transpose(out_fc3_w), ob,
        out_fc3_b[None, :],
    )

    pred_all = pred[None]
    hidden_out = new_h[None]
    return (pred_all, hidden_out)


# speedup vs baseline: 2.5251x; 1.0077x over previous
import functools

import jax
import jax.numpy as jnp
from jax import lax
from jax.experimental import pallas as pl
from jax.experimental.pallas import tpu as pltpu
from jax.experimental.pallas import tpu_sc as plsc

N = 10000
E = 160000
DEG = 16
NH = 128
IN = 4

NC = 2
NS = 16
NW = NC * NS

EPW = E // NW
S1_C = 488
S1_CHUNKS = [(i * S1_C, S1_C) for i in range(EPW // S1_C)]
if EPW % S1_C:
    S1_CHUNKS.append((EPW - EPW % S1_C, EPW % S1_C))

_sc_mesh = plsc.VectorSubcoreMesh(core_axis_name="c", subcore_axis_name="s")


@functools.partial(
    pl.kernel,
    out_type=jax.ShapeDtypeStruct((E, NH), jnp.float32),
    mesh=_sc_mesh,
    scratch_types=[
        pltpu.VMEM((S1_C,), jnp.int32),
        pltpu.VMEM((S1_C,), jnp.int32),
        pltpu.VMEM((S1_C,), jnp.int32),
        pltpu.VMEM((S1_C,), jnp.int32),
        pltpu.VMEM((S1_C, NH), jnp.float32),
        pltpu.VMEM((S1_C, NH), jnp.float32),
        pltpu.SemaphoreType.DMA,
        pltpu.SemaphoreType.DMA,
        pltpu.SemaphoreType.DMA,
        pltpu.SemaphoreType.DMA,
        pltpu.SemaphoreType.DMA,
        pltpu.SemaphoreType.DMA,
    ],
)
def _s1_gather(a_hbm, b_hbm, recv_hbm, send_hbm, out_hbm,
               idx_r0, idx_r1, idx_s0, idx_s1, rows0, rows1,
               si0, si1, sg0, sg1, so0, so1):
    wid = lax.axis_index("s") * NC + lax.axis_index("c")
    base = wid * EPW
    idx_r = (idx_r0, idx_r1)
    idx_s = (idx_s0, idx_s1)
    rows = (rows0, rows1)
    si = (si0, si1)
    sg = (sg0, sg1)
    so = (so0, so1)
    nck = len(S1_CHUNKS)
    d_i = [None] * nck
    d_g = [None] * nck
    d_o = [None] * nck

    def issue_idx(j, s):
        off, c = S1_CHUNKS[j]
        return (
            pltpu.async_copy(recv_hbm.at[pl.ds(base + off, c)],
                             idx_r[s].at[pl.ds(0, c)], si[s]),
            pltpu.async_copy(send_hbm.at[pl.ds(base + off, c)],
                             idx_s[s].at[pl.ds(0, c)], si[s]),
        )

    for j in range(nck):
        s = j % 2
        s2 = 1 - s
        cj = S1_CHUNKS[j][1]
        if j == 0:
            d_i[0] = issue_idx(0, 0)
            d_i[0][0].wait()
            d_i[0][1].wait()
            d_g[0] = pltpu.async_copy(
                a_hbm.at[idx_r[0].at[pl.ds(0, cj)]],
                rows[0].at[pl.ds(0, cj)], sg[0])
            if nck > 1:
                d_i[1] = issue_idx(1, 1)
        d_g[j].wait()
        d_h = pltpu.async_copy(
            b_hbm.at[idx_s[s].at[pl.ds(0, cj)]],
            rows[s].at[pl.ds(0, cj)], sg[s], add=True)
        if j + 1 < nck:
            cn = S1_CHUNKS[j + 1][1]
            if j >= 1:
                d_o[j - 1].wait()
            d_i[j + 1][0].wait()
            d_i[j + 1][1].wait()
            d_g[j + 1] = pltpu.async_copy(
                a_hbm.at[idx_r[s2].at[pl.ds(0, cn)]],
                rows[s2].at[pl.ds(0, cn)], sg[s2])
        d_h.wait()
        if j + 2 < len(S1_CHUNKS):
            d_i[j + 2] = issue_idx(j + 2, s)
        off = S1_CHUNKS[j][0]
        d_o[j] = pltpu.async_copy(
            rows[s].at[pl.ds(0, cj)],
            out_hbm.at[pl.ds(base + off, cj)], so[s])
    if nck >= 2:
        d_o[nck - 2].wait()
    d_o[nck - 1].wait()


NPAD = 10240
NPW = NPAD // NW
S2_Q = 4
S2_QN = NPW // S2_Q


@functools.partial(
    pl.kernel,
    out_type=jax.ShapeDtypeStruct((NPAD, NH), jnp.float32),
    mesh=_sc_mesh,
    scratch_types=[
        pltpu.VMEM((DEG * NPW,), jnp.int32),
        pltpu.VMEM((NPW, NH), jnp.float32),
        pltpu.SemaphoreType.DMA,
        pltpu.SemaphoreType.DMA,
        pltpu.SemaphoreType.DMA,
        pltpu.SemaphoreType.DMA,
    ],
)
def _s2_aggregate(msgs_hbm, e2nw_hbm, out_hbm,
                  idx2, acc, sq0, sq1, sq2, sq3):
    wid = lax.axis_index("s") * NC + lax.axis_index("c")
    nbase = wid * NPW
    sq = (sq0, sq1, sq2, sq3)
    pltpu.sync_copy(e2nw_hbm.at[pl.ds(wid * DEG * NPW, DEG * NPW)], idx2)
    d_prev = [None] * S2_Q
    for k in range(DEG):
        for q in range(S2_Q):
            if k > 0:
                d_prev[q].wait()
            d_prev[q] = pltpu.async_copy(
                msgs_hbm.at[idx2.at[pl.ds(k * NPW + q * S2_QN, S2_QN)]],
                acc.at[pl.ds(q * S2_QN, S2_QN)], sq[q], add=(k > 0))
    for q in range(S2_Q):
        d_prev[q].wait()
    pltpu.sync_copy(acc, out_hbm.at[pl.ds(nbase, NPW)])


P1_BN = 400


def _p1_body(h_ref, w1rT_ref, w1sT_ref, b1_ref, a_ref, b_ref):
    h = h_ref[...]
    a_ref[...] = jnp.dot(h, w1rT_ref[...], preferred_element_type=jnp.float32)
    b_ref[...] = jnp.dot(h, w1sT_ref[...], preferred_element_type=jnp.float32) + b1_ref[...]


def _p1(h, w1rT, w1sT, b1):
    grid = N // P1_BN
    return pl.pallas_call(
        _p1_body,
        grid=(grid,),
        in_specs=[
            pl.BlockSpec((P1_BN, NH), lambda i: (i, 0)),
            pl.BlockSpec((NH, NH), lambda i: (0, 0)),
            pl.BlockSpec((NH, NH), lambda i: (0, 0)),
            pl.BlockSpec((1, NH), lambda i: (0, 0)),
        ],
        out_specs=[
            pl.BlockSpec((P1_BN, NH), lambda i: (i, 0)),
            pl.BlockSpec((P1_BN, NH), lambda i: (i, 0)),
        ],
        out_shape=[
            jax.ShapeDtypeStruct((N, NH), jnp.float32),
            jax.ShapeDtypeStruct((N, NH), jnp.float32),
        ],
    )(h, w1rT, w1sT, b1)


P2_BE = 2000


def _p2_body(pre1_ref, e1_ref, w2T_ref, b2_ref, out_ref):
    msg = jnp.tanh(pre1_ref[...])
    msg = jnp.dot(msg, w2T_ref[...], preferred_element_type=jnp.float32) + b2_ref[...]
    out_ref[...] = jnp.tanh(msg) * e1_ref[...]


def _p2(pre1, e1, w2T, b2):
    grid = E // P2_BE
    return pl.pallas_call(
        _p2_body,
        grid=(grid,),
        in_specs=[
            pl.BlockSpec((P2_BE, NH), lambda i: (i, 0)),
            pl.BlockSpec((P2_BE, 1), lambda i: (i, 0)),
            pl.BlockSpec((NH, NH), lambda i: (0, 0)),
            pl.BlockSpec((1, NH), lambda i: (0, 0)),
        ],
        out_specs=pl.BlockSpec((P2_BE, NH), lambda i: (i, 0)),
        out_shape=jax.ShapeDtypeStruct((E, NH), jnp.float32),
    )(pre1, e1, w2T, b2)


P3_BN = 400


def _p3_body(inc_ref, h_ref, x_ref,
             irT_ref, iiT_ref, inT_ref, ib_ref,
             hrT_ref, hiT_ref, hhT_ref,
             o1T_ref, o2T_ref, o3T_ref, ob_ref, ob3_ref,
             newh_ref, pred_ref):
    agg = inc_ref[...] * (1.0 / float(N - 1))
    x = x_ref[...]
    h = h_ref[...]
    inp_r = jnp.dot(x, irT_ref[...], preferred_element_type=jnp.float32) + ib_ref[0, :1, :]
    inp_i = jnp.dot(x, iiT_ref[...], preferred_element_type=jnp.float32) + ib_ref[0, 1:2, :]
    inp_n = jnp.dot(x, inT_ref[...], preferred_element_type=jnp.float32) + ib_ref[0, 2:3, :]
    r = jax.nn.sigmoid(inp_r + jnp.dot(agg, hrT_ref[...], preferred_element_type=jnp.float32))
    ii = jax.nn.sigmoid(inp_i + jnp.dot(agg, hiT_ref[...], preferred_element_type=jnp.float32))
    nn = jnp.tanh(inp_n + r * jnp.dot(agg, hhT_ref[...], preferred_element_type=jnp.float32))
    new_h = (1.0 - ii) * nn + ii * h
    newh_ref[...] = new_h
    p = jax.nn.relu(jnp.dot(new_h, o1T_ref[...], preferred_element_type=jnp.float32) + ob_ref[0, :1, :])
    p = jax.nn.relu(jnp.dot(p, o2T_ref[...], preferred_element_type=jnp.float32) + ob_ref[0, 1:2, :])
    p3 = jnp.dot(p, o3T_ref[...], preferred_element_type=jnp.float32) + ob3_ref[...]
    pred_ref[...] = x + p3


def _p3(incoming, h, x, irT, iiT, inT, ib, hrT, hiT, hhT, o1T, o2T, o3T, ob, ob3):
    grid = N // P3_BN
    full = lambda i: (0, 0)
    return pl.pallas_call(
        _p3_body,
        grid=(grid,),
        in_specs=[
            pl.BlockSpec((P3_BN, NH), lambda i: (i, 0)),
            pl.BlockSpec((P3_BN, NH), lambda i: (i, 0)),
            pl.BlockSpec((P3_BN, IN), lambda i: (i, 0)),
            pl.BlockSpec((IN, NH), full),
            pl.BlockSpec((IN, NH), full),
            pl.BlockSpec((IN, NH), full),
            pl.BlockSpec((1, 3, NH), lambda i: (0, 0, 0)),
            pl.BlockSpec((NH, NH), full),
            pl.BlockSpec((NH, NH), full),
            pl.BlockSpec((NH, NH), full),
            pl.BlockSpec((NH, NH), full),
            pl.BlockSpec((NH, NH), full),
            pl.BlockSpec((NH, IN), full),
            pl.BlockSpec((1, 2, NH), lambda i: (0, 0, 0)),
            pl.BlockSpec((1, IN), full),
        ],
        out_specs=[
            pl.BlockSpec((P3_BN, NH), lambda i: (i, 0)),
            pl.BlockSpec((P3_BN, IN), lambda i: (i, 0)),
        ],
        out_shape=[
            jax.ShapeDtypeStruct((N, NH), jnp.float32),
            jax.ShapeDtypeStruct((N, IN), jnp.float32),
        ],
    )(incoming, h, x, irT, iiT, inT, ib, hrT, hiT, hhT, o1T, o2T, o3T, ob, ob3)


def kernel(inputs, hidden, edges, node_masks, send_edges, recv_edges,
           edge2node_inds,
           msg_fc1_w, msg_fc1_b, msg_fc2_w, msg_fc2_b,
           hidden_r_w, hidden_i_w, hidden_h_w,
           input_r_w, input_r_b, input_i_w, input_i_b, input_n_w, input_n_b,
           out_fc1_w, out_fc1_b, out_fc2_w, out_fc2_b, out_fc3_w, out_fc3_b):
    h = hidden[0]
    x = inputs[0]
    e1 = edges[0, :, 1:2]

    w1rT = jnp.transpose(msg_fc1_w[1][:, :NH])
    w1sT = jnp.transpose(msg_fc1_w[1][:, NH:])
    b1 = msg_fc1_b[1][None, :]
    w2T = jnp.transpose(msg_fc2_w[1])
    b2 = msg_fc2_b[1][None, :]

    a_tab, b_tab = _p1(h, w1rT, w1sT, b1)

    pre1 = _s1_gather(a_tab, b_tab, recv_edges, send_edges)

    msgs = _p2(pre1, e1, w2T, b2)

    e2nw = (jnp.pad(edge2node_inds, ((0, NPAD - N), (0, 0)))
            .T.reshape(DEG, NW, NPW).transpose(1, 0, 2)
            .reshape(-1))
    incoming = _s2_aggregate(msgs, e2nw)

    ib = jnp.stack([input_r_b, input_i_b, input_n_b])[None]
    ob = jnp.stack([out_fc1_b, out_fc2_b])[None]
    new_h, pred = _p3(
        incoming, h, x,
        jnp.transpose(input_r_w), jnp.transpose(input_i_w), jnp.transpose(input_n_w), ib,
        jnp.transpose(hidden_r_w), jnp.transpose(hidden_i_w), jnp.transpose(hidden_h_w),
        jnp.transpose(out_fc1_w), jnp.transpose(out_fc2_w), jnp.transpose(out_fc3_w), ob,
        out_fc3_b[None, :],
    )

    pred_all = pred[None]
    hidden_out = new_h[None]
    return (pred_all, hidden_out)
